# baseline (speedup 1.0000x reference)
import jax
import jax.numpy as jnp
from jax.experimental import pallas as pl
from jax.experimental.pallas import tpu as pltpu

CH = 2048
NBUF = 3
SUB = 4


def _body(x_hbm, p_hbm, o_hbm, pvmem, xbuf, obuf, xsem, osem, psem):
    N, D = x_hbm.shape
    S = p_hbm.shape[0]
    nch = N // CH
    pch = S // CH
    sc = CH // SUB

    def p_load(j):
        return pltpu.make_async_copy(
            p_hbm.at[pl.ds(j * CH, CH)], pvmem.at[pl.ds(j * CH, CH)], psem.at[j]
        )

    def x_load(i):
        return pltpu.make_async_copy(
            x_hbm.at[pl.ds(i * CH, CH)], xbuf.at[i % NBUF], xsem.at[i % NBUF]
        )

    def o_store(i, s):
        return pltpu.make_async_copy(
            obuf.at[i % NBUF, pl.ds(s * sc, sc)],
            o_hbm.at[pl.ds(i * CH + s * sc, sc)],
            osem.at[i % NBUF, s],
        )

    for j in range(pch):
        p_load(j).start()
    for i in range(NBUF - 1):
        x_load(i).start()

    for i in range(nch):
        if i + NBUF - 1 < nch:
            x_load(i + NBUF - 1).start()
        x_load(i).wait()
        if i < pch:
            p_load(i).wait()
        if i >= NBUF:
            for s in range(SUB):
                o_store(i - NBUF, s).wait()
        prow = (i * CH) % S
        for s in range(SUB):
            obuf[i % NBUF, pl.ds(s * sc, sc)] = (
                xbuf[i % NBUF, pl.ds(s * sc, sc)]
                + pvmem[pl.ds(prow + s * sc, sc), :]
            )
            o_store(i, s).start()

    for i in range(max(nch - NBUF, 0), nch):
        for s in range(SUB):
            o_store(i, s).wait()


def kernel(inputs, pos_table):
    B, S, D = inputs.shape
    x = inputs.reshape(B * S, D)
    out = pl.pallas_call(
        _body,
        in_specs=[
            pl.BlockSpec(memory_space=pl.ANY),
            pl.BlockSpec(memory_space=pl.ANY),
        ],
        out_specs=pl.BlockSpec(memory_space=pl.ANY),
        out_shape=jax.ShapeDtypeStruct((B * S, D), inputs.dtype),
        scratch_shapes=[
            pltpu.VMEM((S, D), jnp.float32),
            pltpu.VMEM((NBUF, CH, D), jnp.float32),
            pltpu.VMEM((NBUF, CH, D), jnp.float32),
            pltpu.SemaphoreType.DMA((NBUF,)),
            pltpu.SemaphoreType.DMA((NBUF, SUB)),
            pltpu.SemaphoreType.DMA((S // CH,)),
        ],
    )(x, pos_table)
    return out.reshape(B, S, D)


# speedup vs baseline: 1.0039x; 1.0039x over previous
import jax
import jax.numpy as jnp
from jax.experimental import pallas as pl
from jax.experimental.pallas import tpu as pltpu

CH = 2048
NBUF = 3


def _body(x_hbm, p_hbm, o_hbm, pvmem, xbuf, obuf, xsem, osem, psem):
    N, D = x_hbm.shape
    S = p_hbm.shape[0]
    nch = N // CH
    pch = S // CH

    def p_load(j):
        return pltpu.make_async_copy(
            p_hbm.at[pl.ds(j * CH, CH)], pvmem.at[pl.ds(j * CH, CH)], psem.at[j]
        )

    def x_load(i):
        return pltpu.make_async_copy(
            x_hbm.at[pl.ds(i * CH, CH)], xbuf.at[i % NBUF], xsem.at[i % NBUF]
        )

    def o_store(i):
        return pltpu.make_async_copy(
            obuf.at[i % NBUF], o_hbm.at[pl.ds(i * CH, CH)], osem.at[i % NBUF]
        )

    for j in range(pch):
        p_load(j).start()
    for i in range(NBUF - 1):
        x_load(i).start()

    for i in range(nch):
        if i + NBUF - 1 < nch:
            x_load(i + NBUF - 1).start()
        x_load(i).wait()
        if i < pch:
            p_load(i).wait()
        if i >= NBUF:
            o_store(i - NBUF).wait()
        prow = (i * CH) % S
        obuf[i % NBUF] = xbuf[i % NBUF] + pvmem[pl.ds(prow, CH), :]
        o_store(i).start()

    for i in range(max(nch - NBUF, 0), nch):
        o_store(i).wait()


def kernel(inputs, pos_table):
    B, S, D = inputs.shape
    x = inputs.reshape(B * S, D)
    out = pl.pallas_call(
        _body,
        in_specs=[
            pl.BlockSpec(memory_space=pl.ANY),
            pl.BlockSpec(memory_space=pl.ANY),
        ],
        out_specs=pl.BlockSpec(memory_space=pl.ANY),
        out_shape=jax.ShapeDtypeStruct((B * S, D), inputs.dtype),
        scratch_shapes=[
            pltpu.VMEM((S, D), jnp.float32),
            pltpu.VMEM((NBUF, CH, D), jnp.float32),
            pltpu.VMEM((NBUF, CH, D), jnp.float32),
            pltpu.SemaphoreType.DMA((NBUF,)),
            pltpu.SemaphoreType.DMA((NBUF,)),
            pltpu.SemaphoreType.DMA((S // CH,)),
        ],
    )(x, pos_table)
    return out.reshape(B, S, D)
